# baseline (speedup 1.0000x reference)
import functools

import jax
import jax.numpy as jnp
from jax import lax
from jax.experimental import pallas as pl
from jax.experimental.pallas import tpu as pltpu
from jax.experimental.pallas import tpu_sc as plsc

V = 8192
B = 4096
NC = 2
NS = 16
NW = NC * NS
RPW = B // NW
C = 4
NCHUNK = RPW // C
L = 16


UNROLL = 16


def _row_sumexp(rows_ref, r):
    zero = jnp.zeros((L,), jnp.float32)

    def body(j, accs):
        base = j * (UNROLL * L)
        vs = [rows_ref[r, pl.ds(base + u * L, L)] for u in range(UNROLL)]
        return tuple(a + jnp.exp(v) for a, v in zip(accs, vs))

    accs = lax.fori_loop(0, V // (UNROLL * L), body, (zero,) * UNROLL)
    total = accs[0]
    for a in accs[1:]:
        total = total + a
    return total


def _sc_body(x2_hbm, tf_hbm, table_hbm,
             out_logits, out_sums, out_picked,
             xid_v, tf_v, rows0, rows1, sums_v, picked_v,
             g0, g1, s0, s1):
    wid = lax.axis_index("s") * NC + lax.axis_index("c")
    pltpu.sync_copy(x2_hbm.at[pl.ds(wid * NCHUNK, NCHUNK)], xid_v)
    pltpu.sync_copy(tf_hbm.at[pl.ds(wid * RPW, RPW)], tf_v)

    rows = (rows0, rows1)
    gsem = (g0, g1)
    ssem = (s0, s1)
    iota = lax.broadcasted_iota(jnp.int32, (L,), 0)
    lane_r = iota & (C - 1)
    lane_mask = iota < C

    def gather(c, b):
        return pltpu.make_async_copy(
            table_hbm.at[xid_v.at[c]], rows[b], gsem[b])

    def scatter_cp(c, b):
        return pltpu.make_async_copy(
            rows[b], out_logits.at[pl.ds(wid * RPW + c * C, C)], ssem[b])

    gather(0, 0).start()

    def outer(k, pick_acc):
        tq16 = tf_v[pl.ds(pl.multiple_of(k * L, L), L)]
        for j in range(C):
            c = k * C + j
            b = j % 2
            nb = 1 - b
            if j == 0:
                @pl.when(k >= 1)
                def _():
                    scatter_cp(c - 1, nb).wait()
            else:
                scatter_cp(c - 1, nb).wait()
            if j == C - 1:
                @pl.when(k < NCHUNK // C - 1)
                def _():
                    gather(c + 1, nb).start()
            else:
                gather(c + 1, nb).start()
            gather(c, b).wait()

            for r in range(C):
                sums_v[c * C + r, :] = _row_sumexp(rows[b], r)

            for r in range(C):
                t = tq16[j * C + r]
                v = rows[b][r, pl.ds(pl.multiple_of(t & ~(L - 1), L), L)]
                pick_acc = pick_acc + jnp.where(
                    iota == (t & (L - 1)), v, jnp.zeros((L,), jnp.float32))

            scatter_cp(c, b).start()
        return pick_acc

    pick_acc = lax.fori_loop(0, NCHUNK // C, outer, jnp.zeros((L,), jnp.float32))
    scatter_cp(NCHUNK - 1, 1).wait()
    picked_v[...] = pick_acc
    pltpu.sync_copy(sums_v, out_sums.at[pl.ds(wid * RPW, RPW)])
    pltpu.sync_copy(picked_v, out_picked.at[wid])


_bigram_sc = functools.partial(
    pl.kernel,
    mesh=plsc.VectorSubcoreMesh(core_axis_name="c", subcore_axis_name="s"),
    out_type=(
        jax.ShapeDtypeStruct((B, V), jnp.float32),
        jax.ShapeDtypeStruct((B, L), jnp.float32),
        jax.ShapeDtypeStruct((NW, L), jnp.float32),
    ),
    scratch_types=[
        pltpu.VMEM((NCHUNK, C), jnp.int32),
        pltpu.VMEM((RPW,), jnp.int32),
        pltpu.VMEM((C, V), jnp.float32),
        pltpu.VMEM((C, V), jnp.float32),
        pltpu.VMEM((RPW, L), jnp.float32),
        pltpu.VMEM((L,), jnp.float32),
        pltpu.SemaphoreType.DMA,
        pltpu.SemaphoreType.DMA,
        pltpu.SemaphoreType.DMA,
        pltpu.SemaphoreType.DMA,
    ],
)(_sc_body)


def kernel(x_ids, targets, logits_table):
    x2 = x_ids.reshape(B // C, C)
    logits, sum_parts, pick_parts = _bigram_sc(x2, targets, logits_table)
    loss = jnp.mean(jnp.log(sum_parts.sum(axis=1))) - pick_parts.sum() / B
    return logits, loss


# speedup vs baseline: 1.0124x; 1.0124x over previous
import functools

import jax
import jax.numpy as jnp
from jax import lax
from jax.experimental import pallas as pl
from jax.experimental.pallas import tpu as pltpu
from jax.experimental.pallas import tpu_sc as plsc

V = 8192
B = 4096
NC = 2
NS = 16
NW = NC * NS
RPW = B // NW
C = 4
NCHUNK = RPW // C
L = 16


UNROLL = 8


def _row_sumexp(rows_ref, r):
    zero = jnp.zeros((L,), jnp.float32)

    def body(j, accs):
        base = j * (UNROLL * L)
        vs = [rows_ref[r, pl.ds(base + u * L, L)] for u in range(UNROLL)]
        return tuple(a + jnp.exp(v) for a, v in zip(accs, vs))

    accs = lax.fori_loop(0, V // (UNROLL * L), body, (zero,) * UNROLL)
    total = accs[0]
    for a in accs[1:]:
        total = total + a
    return total


NBUF = 3


def _sc_body(x2_hbm, tp_hbm, table_hbm,
             out_logits, out_sums, out_picked,
             xid_v, tp_v, rows0, rows1, rows2, sums_v, picked_v,
             g0, g1, g2, s0, s1, s2):
    wid = lax.axis_index("s") * NC + lax.axis_index("c")
    pltpu.sync_copy(x2_hbm.at[pl.ds(wid * NCHUNK, NCHUNK)], xid_v)
    pltpu.sync_copy(tp_hbm.at[pl.ds(wid * NCHUNK * L, NCHUNK * L)], tp_v)

    rows = (rows0, rows1, rows2)
    gsem = (g0, g1, g2)
    ssem = (s0, s1, s2)
    iota = lax.broadcasted_iota(jnp.int32, (L,), 0)

    def gather(c, b):
        return pltpu.make_async_copy(
            table_hbm.at[xid_v.at[c]], rows[b], gsem[b])

    def scatter_cp(c, b):
        return pltpu.make_async_copy(
            rows[b], out_logits.at[pl.ds(wid * RPW + c * C, C)], ssem[b])

    def chunk_step(c, b, pick_acc, skip_wait=False, skip_start=False):
        if not skip_wait:
            scatter_cp(c - 2, (b + 1) % NBUF).wait()
        if not skip_start:
            gather(c + 1, (b + 1) % NBUF).start()
        gather(c, b).wait()

        for r in range(C):
            sums_v[c * C + r, :] = _row_sumexp(rows[b], r)

        tq = tp_v[pl.ds(pl.multiple_of(c * L, L), L)]
        for r in range(C):
            t = tq[r]
            v = rows[b][r, pl.ds(pl.multiple_of(t & ~(L - 1), L), L)]
            pick_acc = pick_acc + jnp.where(
                iota == (t & (L - 1)), v, jnp.zeros((L,), jnp.float32))

        scatter_cp(c, b).start()
        return pick_acc

    gather(0, 0).start()

    def outer(k, pick_acc):
        for j in range(NBUF):
            c = k * NBUF + j
            if j == NBUF - 1:
                pick_acc = chunk_step(c, j, pick_acc)
            else:
                @pl.when(k >= 1)
                def _():
                    scatter_cp(c - 2, (j + 1) % NBUF).wait()
                pick_acc = chunk_step(c, j, pick_acc, skip_wait=True)
        return pick_acc

    nfull = (NCHUNK - 2) // NBUF
    pick_acc = lax.fori_loop(0, nfull, outer, jnp.zeros((L,), jnp.float32))
    pick_acc = chunk_step(NCHUNK - 2, (NCHUNK - 2) % NBUF, pick_acc)
    pick_acc = chunk_step(NCHUNK - 1, (NCHUNK - 1) % NBUF, pick_acc,
                          skip_start=True)
    scatter_cp(NCHUNK - 2, (NCHUNK - 2) % NBUF).wait()
    scatter_cp(NCHUNK - 1, (NCHUNK - 1) % NBUF).wait()
    picked_v[...] = pick_acc
    pltpu.sync_copy(sums_v, out_sums.at[pl.ds(wid * RPW, RPW)])
    pltpu.sync_copy(picked_v, out_picked.at[wid])


_bigram_sc = functools.partial(
    pl.kernel,
    mesh=plsc.VectorSubcoreMesh(core_axis_name="c", subcore_axis_name="s"),
    out_type=(
        jax.ShapeDtypeStruct((B, V), jnp.float32),
        jax.ShapeDtypeStruct((B, L), jnp.float32),
        jax.ShapeDtypeStruct((NW, L), jnp.float32),
    ),
    scratch_types=[
        pltpu.VMEM((NCHUNK, C), jnp.int32),
        pltpu.VMEM((NCHUNK * L,), jnp.int32),
        pltpu.VMEM((C, V), jnp.float32),
        pltpu.VMEM((C, V), jnp.float32),
        pltpu.VMEM((C, V), jnp.float32),
        pltpu.VMEM((RPW, L), jnp.float32),
        pltpu.VMEM((L,), jnp.float32),
        pltpu.SemaphoreType.DMA,
        pltpu.SemaphoreType.DMA,
        pltpu.SemaphoreType.DMA,
        pltpu.SemaphoreType.DMA,
        pltpu.SemaphoreType.DMA,
        pltpu.SemaphoreType.DMA,
    ],
)(_sc_body)


def kernel(x_ids, targets, logits_table):
    x2 = x_ids.reshape(B // C, C)
    tpad = jnp.pad(targets.reshape(B // C, C), ((0, 0), (0, L - C)))
    logits, sum_parts, pick_parts = _bigram_sc(
        x2, tpad.reshape(-1), logits_table)
    loss = jnp.mean(jnp.log(sum_parts.sum(axis=1))) - pick_parts.sum() / B
    return logits, loss


# speedup vs baseline: 1.0323x; 1.0196x over previous
import functools

import jax
import jax.numpy as jnp
from jax import lax
from jax.experimental import pallas as pl
from jax.experimental.pallas import tpu as pltpu
from jax.experimental.pallas import tpu_sc as plsc

V = 8192
B = 4096
NC = 2
NS = 16
NW = NC * NS
RPW = B // NW
C = 4
NCHUNK = RPW // C
L = 16


UNROLL = 8


def _row_sumexp(rows_ref, r):
    zero = jnp.zeros((L,), jnp.float32)

    def body(j, accs):
        base = j * (UNROLL * L)
        vs = [rows_ref[r, pl.ds(base + u * L, L)] for u in range(UNROLL)]
        return tuple(a + jnp.exp(v) for a, v in zip(accs, vs))

    accs = lax.fori_loop(0, V // (UNROLL * L), body, (zero,) * UNROLL)
    total = accs[0]
    for a in accs[1:]:
        total = total + a
    return total


NBUF = 3


def _sc_body(x2_hbm, tp_hbm, table_hbm,
             out_logits, out_sums, out_picked,
             xid_v, tp_v, rows0, rows1, rows2, sums_v, picked_v,
             g0, g1, g2, s0, s1, s2):
    wid = lax.axis_index("s") * NC + lax.axis_index("c")
    pltpu.sync_copy(x2_hbm.at[pl.ds(wid * NCHUNK, NCHUNK)], xid_v)
    pltpu.sync_copy(tp_hbm.at[pl.ds(wid * NCHUNK * L, NCHUNK * L)], tp_v)

    rows = (rows0, rows1, rows2)
    gsem = (g0, g1, g2)
    ssem = (s0, s1, s2)
    iota = lax.broadcasted_iota(jnp.int32, (L,), 0)

    def gather(c, b):
        return pltpu.make_async_copy(
            table_hbm.at[xid_v.at[c]], rows[b], gsem[b])

    def scatter_cp(c, b):
        return pltpu.make_async_copy(
            rows[b], out_logits.at[pl.ds(wid * RPW + c * C, C)], ssem[b])

    def chunk_step(c, b, pick_acc, skip_wait=False, skip_start=False):
        if not skip_wait:
            scatter_cp(c - 2, (b + 1) % NBUF).wait()
        if not skip_start:
            gather(c + 1, (b + 1) % NBUF).start()
        gather(c, b).wait()
        scatter_cp(c, b).start()

        for r in range(C):
            sums_v[c * C + r, :] = _row_sumexp(rows[b], r)

        tq = tp_v[pl.ds(pl.multiple_of(c * L, L), L)]
        for r in range(C):
            t = tq[r]
            v = rows[b][r, pl.ds(pl.multiple_of(t & ~(L - 1), L), L)]
            pick_acc = pick_acc + jnp.where(
                iota == (t & (L - 1)), v, jnp.zeros((L,), jnp.float32))

        return pick_acc

    gather(0, 0).start()

    def outer(k, pick_acc):
        for j in range(NBUF):
            c = k * NBUF + j
            if j == NBUF - 1:
                pick_acc = chunk_step(c, j, pick_acc)
            else:
                @pl.when(k >= 1)
                def _():
                    scatter_cp(c - 2, (j + 1) % NBUF).wait()
                pick_acc = chunk_step(c, j, pick_acc, skip_wait=True)
        return pick_acc

    nfull = (NCHUNK - 2) // NBUF
    pick_acc = lax.fori_loop(0, nfull, outer, jnp.zeros((L,), jnp.float32))
    pick_acc = chunk_step(NCHUNK - 2, (NCHUNK - 2) % NBUF, pick_acc)
    pick_acc = chunk_step(NCHUNK - 1, (NCHUNK - 1) % NBUF, pick_acc,
                          skip_start=True)
    scatter_cp(NCHUNK - 2, (NCHUNK - 2) % NBUF).wait()
    scatter_cp(NCHUNK - 1, (NCHUNK - 1) % NBUF).wait()
    picked_v[...] = pick_acc
    pltpu.sync_copy(sums_v, out_sums.at[pl.ds(wid * RPW, RPW)])
    pltpu.sync_copy(picked_v, out_picked.at[wid])


_bigram_sc = functools.partial(
    pl.kernel,
    mesh=plsc.VectorSubcoreMesh(core_axis_name="c", subcore_axis_name="s"),
    out_type=(
        jax.ShapeDtypeStruct((B, V), jnp.float32),
        jax.ShapeDtypeStruct((B, L), jnp.float32),
        jax.ShapeDtypeStruct((NW, L), jnp.float32),
    ),
    scratch_types=[
        pltpu.VMEM((NCHUNK, C), jnp.int32),
        pltpu.VMEM((NCHUNK * L,), jnp.int32),
        pltpu.VMEM((C, V), jnp.float32),
        pltpu.VMEM((C, V), jnp.float32),
        pltpu.VMEM((C, V), jnp.float32),
        pltpu.VMEM((RPW, L), jnp.float32),
        pltpu.VMEM((L,), jnp.float32),
        pltpu.SemaphoreType.DMA,
        pltpu.SemaphoreType.DMA,
        pltpu.SemaphoreType.DMA,
        pltpu.SemaphoreType.DMA,
        pltpu.SemaphoreType.DMA,
        pltpu.SemaphoreType.DMA,
    ],
)(_sc_body)


def kernel(x_ids, targets, logits_table):
    x2 = x_ids.reshape(B // C, C)
    tpad = jnp.pad(targets.reshape(B // C, C), ((0, 0), (0, L - C)))
    logits, sum_parts, pick_parts = _bigram_sc(
        x2, tpad.reshape(-1), logits_table)
    loss = jnp.mean(jnp.log(sum_parts.sum(axis=1))) - pick_parts.sum() / B
    return logits, loss
